# Initial kernel scaffold; baseline (speedup 1.0000x reference)
import jax
import jax.numpy as jnp
from jax.experimental import pallas as pl


def kernel(x, Wp1, bp1, Wn1, Ws1, bs1, Wp2, bp2, Wn2, Ws2, bs2, W3, b3, W4, b4, edge_index):
    raise NotImplementedError("write your pallas kernel here")


# speedup vs baseline: 1.0457x; 1.0457x over previous
import functools

import jax
import jax.numpy as jnp
from jax.experimental import pallas as pl
from jax.experimental.pallas import tpu as pltpu

N = 10000
D = 128
ROWS_PER_BLK = 2000


def _dense_body(x_ref, w_ref, b_ref, o_ref, *, act):
    h = jnp.dot(x_ref[...], w_ref[...], preferred_element_type=jnp.float32)
    h = h + b_ref[...]
    if act == "relu":
        h = jnp.maximum(h, 0.0)
    elif act == "leaky":
        h = jnp.where(h >= 0.0, h, 0.01 * h)
    o_ref[...] = h


def _dense(x, w, b, act):
    n, d = x.shape
    dout = w.shape[1]
    grid = (n // ROWS_PER_BLK,)
    return pl.pallas_call(
        functools.partial(_dense_body, act=act),
        grid=grid,
        in_specs=[
            pl.BlockSpec((ROWS_PER_BLK, d), lambda i: (i, 0)),
            pl.BlockSpec((d, dout), lambda i: (0, 0)),
            pl.BlockSpec((dout,), lambda i: (0,)),
        ],
        out_specs=pl.BlockSpec((ROWS_PER_BLK, dout), lambda i: (i, 0)),
        out_shape=jax.ShapeDtypeStruct((n, dout), jnp.float32),
    )(x, w, b)


def _sage_tail_body(x_ref, ws_ref, bs_ref, agg_ref, wn_ref, o_ref):
    h = jnp.dot(x_ref[...], ws_ref[...], preferred_element_type=jnp.float32)
    h = h + bs_ref[...]
    h = h + jnp.dot(agg_ref[...], wn_ref[...], preferred_element_type=jnp.float32)
    o_ref[...] = jnp.where(h >= 0.0, h, 0.01 * h)


def _sage_tail(x, ws, bs, agg, wn):
    n, d = x.shape
    dout = wn.shape[1]
    grid = (n // ROWS_PER_BLK,)
    return pl.pallas_call(
        _sage_tail_body,
        grid=grid,
        in_specs=[
            pl.BlockSpec((ROWS_PER_BLK, d), lambda i: (i, 0)),
            pl.BlockSpec((d, dout), lambda i: (0, 0)),
            pl.BlockSpec((dout,), lambda i: (0,)),
            pl.BlockSpec((ROWS_PER_BLK, d), lambda i: (i, 0)),
            pl.BlockSpec((d, dout), lambda i: (0, 0)),
        ],
        out_specs=pl.BlockSpec((ROWS_PER_BLK, dout), lambda i: (i, 0)),
        out_shape=jax.ShapeDtypeStruct((n, dout), jnp.float32),
    )(x, ws, bs, agg, wn)


def _head_body(x_ref, w3_ref, b3_ref, w4_ref, b4_ref, o_ref):
    h = jnp.dot(x_ref[...], w3_ref[...], preferred_element_type=jnp.float32)
    h = h + b3_ref[...]
    h = jnp.where(h >= 0.0, h, 0.01 * h)
    h = jnp.dot(h, w4_ref[...], preferred_element_type=jnp.float32)
    h = h + b4_ref[...]
    o_ref[...] = jax.nn.sigmoid(h)


def _head(x, w3, b3, w4, b4):
    n, d = x.shape
    c = w4.shape[1]
    grid = (n // ROWS_PER_BLK,)
    return pl.pallas_call(
        _head_body,
        grid=grid,
        in_specs=[
            pl.BlockSpec((ROWS_PER_BLK, d), lambda i: (i, 0)),
            pl.BlockSpec((d, d), lambda i: (0, 0)),
            pl.BlockSpec((d,), lambda i: (0,)),
            pl.BlockSpec((d, c), lambda i: (0, 0)),
            pl.BlockSpec((c,), lambda i: (0,)),
        ],
        out_specs=pl.BlockSpec((ROWS_PER_BLK, c), lambda i: (i, 0)),
        out_shape=jax.ShapeDtypeStruct((n, c), jnp.float32),
    )(x, w3, b3, w4, b4)


def _segmax(hpool, src, dst):
    msg = jnp.take(hpool, src, axis=0)
    agg = jax.ops.segment_max(msg, dst, num_segments=N)
    return jnp.where(jnp.isfinite(agg), agg, 0.0)


def kernel(x, Wp1, bp1, Wn1, Ws1, bs1, Wp2, bp2, Wn2, Ws2, bs2, W3, b3, W4, b4, edge_index):
    src = edge_index[0]
    dst = edge_index[1]
    hp1 = _dense(x, Wp1, bp1, "relu")
    agg1 = _segmax(hp1, src, dst)
    h1 = _sage_tail(x, Ws1, bs1, agg1, Wn1)
    hp2 = _dense(h1, Wp2, bp2, "relu")
    agg2 = _segmax(hp2, src, dst)
    h2 = _sage_tail(h1, Ws2, bs2, agg2, Wn2)
    return _head(h2, W3, b3, W4, b4)
